# baseline (speedup 1.0000x reference)
import jax
import jax.numpy as jnp
from jax import lax
from jax.experimental import pallas as pl
from jax.experimental.pallas import tpu as pltpu
from jax.experimental.pallas import tpu_sc as plsc

_N_NODES = 50000
_N_EDGES = 1600000
_ALPHA = 0.25
_NC, _NS, _L = 2, 16, 16
_NW = _NC * _NS
_BLK = 128
_NBLK = _N_EDGES // _BLK
_BPW = _NBLK // _NW
_NREM = _NBLK - _BPW * _NW
_NCHUNK = 5
_CBLK = _BPW // _NCHUNK
_CW = _CBLK * _BLK
_CVECS = _CW // _L
_TBL_VECS = _N_NODES // _L


def _focal_acc(a, cs, cd, x, si, di, k):
    tm = jnp.logical_and(
        jnp.logical_and(cs == cd, cs != 0),
        (si < k) == (di < k))
    pos = x >= 0.0
    m2 = jnp.logical_xor(tm, pos)
    ax = jnp.abs(x)
    u = jnp.exp(-ax)
    a1 = 1.0 + u
    b1 = 2.0 + u
    r = 1.0 / (a1 * b1)
    inv = b1 * r
    z = (u * a1) * r
    z2 = z * z
    poly = 1.0 + z2 * (1.0 / 3.0 + z2 * (1.0 / 5.0 + z2 * (1.0 / 7.0 + z2 * (1.0 / 9.0))))
    l1p = (z + z) * poly
    ce = jnp.where(m2, ax, 0.0) + l1p
    q = jnp.where(m2, inv, u * inv)
    at = jnp.where(tm, _ALPHA, 1.0 - _ALPHA)
    return a + (at * ce) * (q * q)


def _make_step(code_v, ev, lv, k):
    def step(i, a):
        for j in range(_BLK // _L):
            sl = pl.ds(i * _BLK + j * _L, _L)
            si = ev[0, sl]
            di = ev[1, sl]
            cs = plsc.load_gather(code_v, [si])
            cd = plsc.load_gather(code_v, [di])
            a = _focal_acc(a, cs, cd, lv[sl], si, di, k)
        return a

    return step


def _body(ei_hbm, x_hbm, batch_hbm, pi_hbm, out_hbm,
          code_v, e0, l0, e1, l1, ex_e, ex_l, res_v, sem0, sem1):
    wid = lax.axis_index("s") * _NC + lax.axis_index("c")
    base_c = wid * _BPW * _BLK
    slots = ((e0, l0, sem0), (e1, l1, sem1))

    pltpu.sync_copy(batch_hbm, code_v)

    def count(i, c):
        return c + code_v[pl.ds(i * _L, _L)]

    ones = plsc.parallel_loop(
        0, _TBL_VECS, 1, unroll=5, carry=jnp.zeros((_L,), jnp.int32))(count)
    k = _N_NODES - jax.lax.reduce_sum(ones, axes=(0,))

    pltpu.sync_copy(pi_hbm, code_v)

    def start(c, slot):
        ev, lv, sem = slots[slot]
        off = base_c + c * _CW
        return (
            pltpu.async_copy(ei_hbm.at[:, pl.ds(off, _CW)], ev, sem),
            pltpu.async_copy(x_hbm.at[pl.ds(off, _CW)], lv, sem),
        )

    acc = jnp.zeros((_L,), jnp.float32)
    inflight = {0: start(0, 0)}
    for c in range(_NCHUNK):
        if c + 1 < _NCHUNK:
            inflight[c + 1] = start(c + 1, (c + 1) % 2)
        for h in inflight.pop(c):
            h.wait()
        ev, lv, _ = slots[c % 2]
        acc = plsc.parallel_loop(0, _CBLK, 1, unroll=1, carry=acc)(
            _make_step(code_v, ev, lv, k))

    @pl.when(wid < _NREM)
    def _extra():
        off = (_NBLK - _NREM + wid) * _BLK
        pltpu.sync_copy(ei_hbm.at[:, pl.ds(off, _BLK)], ex_e)
        pltpu.sync_copy(x_hbm.at[pl.ds(off, _BLK)], ex_l)
        a = _make_step(code_v, ex_e, ex_l, k)(0, jnp.zeros((_L,), jnp.float32))
        res_v[:] = a
    @pl.when(wid >= _NREM)
    def _noextra():
        res_v[:] = jnp.zeros((_L,), jnp.float32)

    res_v[:] = (res_v[:] + acc) * (1.0 / _N_EDGES)
    pltpu.sync_copy(res_v, out_hbm.at[wid])


def kernel(edge_logits, node_logits, edge_index, batch, point_instances):
    del node_logits
    ei = edge_index.astype(jnp.int32)
    x = edge_logits.reshape(-1).astype(jnp.float32)
    b = batch.astype(jnp.int32)
    pi = point_instances.astype(jnp.int32)

    mesh = plsc.VectorSubcoreMesh(core_axis_name="c", subcore_axis_name="s")
    out = pl.kernel(
        _body,
        out_type=jax.ShapeDtypeStruct((_NW, _L), jnp.float32),
        mesh=mesh,
        compiler_params=pltpu.CompilerParams(needs_layout_passes=False),
        scratch_types=[
            pltpu.VMEM((_N_NODES,), jnp.int32),
            pltpu.VMEM((2, _CW), jnp.int32),
            pltpu.VMEM((_CW,), jnp.float32),
            pltpu.VMEM((2, _CW), jnp.int32),
            pltpu.VMEM((_CW,), jnp.float32),
            pltpu.VMEM((2, _BLK), jnp.int32),
            pltpu.VMEM((_BLK,), jnp.float32),
            pltpu.VMEM((_L,), jnp.float32),
            pltpu.SemaphoreType.DMA,
            pltpu.SemaphoreType.DMA,
        ],
    )(ei, x, b, pi)
    return jnp.sum(out)


# speedup vs baseline: 1.2775x; 1.2775x over previous
import functools

import jax
import jax.numpy as jnp
from jax import lax
from jax.experimental import pallas as pl
from jax.experimental.pallas import tpu as pltpu
from jax.experimental.pallas import tpu_sc as plsc

_N_NODES = 50000
_N_EDGES = 1600000
_ALPHA = 0.25
_NC, _NS, _L = 2, 16, 16
_NW = _NC * _NS
_BLK = 128
_NBLK = _N_EDGES // _BLK
_BPW = _NBLK // _NW
_NREM = _NBLK - _BPW * _NW
_NCHUNK = 5
_CBLK = _BPW // _NCHUNK
_CW = _CBLK * _BLK
_CVECS = _CW // _L
_TBL_VECS = _N_NODES // _L

_TC_ROWS = 100
_TC_GRID = _NBLK // _TC_ROWS


def _tc_body(x_ref, d_ref, s_ref):
    x = x_ref[...]
    ax = jnp.abs(x)
    l1p = jnp.log1p(jnp.exp(-ax))
    ce0 = jnp.maximum(x, 0.0) + l1p
    ce1 = ce0 - x
    p = jax.nn.sigmoid(x)
    f0 = ((1.0 - _ALPHA) * ce0) * (p * p)
    omp = 1.0 - p
    f1 = (_ALPHA * ce1) * (omp * omp)
    d_ref[...] = f1 - f0
    s_ref[...] = jnp.sum(f0, axis=0, keepdims=True) * (1.0 / _N_EDGES)


def _make_step(code_v, ev, dv, k):
    def step(i, a):
        sl = pl.ds(i * _L, _L)
        si = ev[0, sl]
        di = ev[1, sl]
        cs = plsc.load_gather(code_v, [si])
        cd = plsc.load_gather(code_v, [di])
        tm = jnp.logical_and(
            jnp.logical_and(cs == cd, cs != 0),
            (si < k) == (di < k))
        return a + jnp.where(tm, dv[sl], 0.0)

    return step


def _sc_body(ei_hbm, d_hbm, batch_hbm, pi_hbm, out_hbm,
             code_v, e0, d0, e1, d1, ex_e, ex_d, res_v, sem0, sem1):
    wid = lax.axis_index("s") * _NC + lax.axis_index("c")
    base_c = wid * _BPW * _BLK
    slots = ((e0, d0, sem0), (e1, d1, sem1))

    pltpu.sync_copy(batch_hbm, code_v)

    def count(i, c):
        return c + code_v[pl.ds(i * _L, _L)]

    ones = plsc.parallel_loop(
        0, _TBL_VECS, 1, unroll=5, carry=jnp.zeros((_L,), jnp.int32))(count)
    k = _N_NODES - jax.lax.reduce_sum(ones, axes=(0,))

    pltpu.sync_copy(pi_hbm, code_v)

    def start(c, slot):
        ev, dv, sem = slots[slot]
        off = base_c + c * _CW
        return (
            pltpu.async_copy(ei_hbm.at[:, pl.ds(off, _CW)], ev, sem),
            pltpu.async_copy(d_hbm.at[pl.ds(off, _CW)], dv, sem),
        )

    acc = jnp.zeros((_L,), jnp.float32)
    inflight = {0: start(0, 0)}
    for c in range(_NCHUNK):
        if c + 1 < _NCHUNK:
            inflight[c + 1] = start(c + 1, (c + 1) % 2)
        for h in inflight.pop(c):
            h.wait()
        ev, dv, _ = slots[c % 2]
        acc = plsc.parallel_loop(0, _CVECS, 1, unroll=8, carry=acc)(
            _make_step(code_v, ev, dv, k))

    @pl.when(wid < _NREM)
    def _extra():
        off = (_NBLK - _NREM + wid) * _BLK
        pltpu.sync_copy(ei_hbm.at[:, pl.ds(off, _BLK)], ex_e)
        pltpu.sync_copy(d_hbm.at[pl.ds(off, _BLK)], ex_d)
        a = lax.fori_loop(
            0, _BLK // _L,
            _make_step(code_v, ex_e, ex_d, k),
            jnp.zeros((_L,), jnp.float32))
        res_v[:] = a

    @pl.when(wid >= _NREM)
    def _noextra():
        res_v[:] = jnp.zeros((_L,), jnp.float32)

    res_v[:] = (res_v[:] + acc) * (1.0 / _N_EDGES)
    pltpu.sync_copy(res_v, out_hbm.at[wid])


def kernel(edge_logits, node_logits, edge_index, batch, point_instances):
    del node_logits
    ei = edge_index.astype(jnp.int32)
    x2 = edge_logits.reshape(_NBLK, _BLK).astype(jnp.float32)
    b = batch.astype(jnp.int32)
    pi = point_instances.astype(jnp.int32)

    d2, s0 = pl.pallas_call(
        _tc_body,
        out_shape=[
            jax.ShapeDtypeStruct((_NBLK, _BLK), jnp.float32),
            jax.ShapeDtypeStruct((1, _BLK), jnp.float32),
        ],
    )(x2)

    mesh = plsc.VectorSubcoreMesh(core_axis_name="c", subcore_axis_name="s")
    out = pl.kernel(
        _sc_body,
        out_type=jax.ShapeDtypeStruct((_NW, _L), jnp.float32),
        mesh=mesh,
        compiler_params=pltpu.CompilerParams(needs_layout_passes=False),
        scratch_types=[
            pltpu.VMEM((_N_NODES,), jnp.int32),
            pltpu.VMEM((2, _CW), jnp.int32),
            pltpu.VMEM((_CW,), jnp.float32),
            pltpu.VMEM((2, _CW), jnp.int32),
            pltpu.VMEM((_CW,), jnp.float32),
            pltpu.VMEM((2, _BLK), jnp.int32),
            pltpu.VMEM((_BLK,), jnp.float32),
            pltpu.VMEM((_L,), jnp.float32),
            pltpu.SemaphoreType.DMA,
            pltpu.SemaphoreType.DMA,
        ],
    )(ei, d2.reshape(-1), b, pi)
    return jnp.sum(out) + jnp.sum(s0)
